# baseline (speedup 1.0000x reference)
import functools

import jax
import jax.numpy as jnp
from jax import lax
from jax.experimental import pallas as pl
from jax.experimental.pallas import tpu as pltpu
from jax.experimental.pallas import tpu_sc as plsc

_N_WORKERS = 32
_ROWS_PER_TILE = 8
_LA = 128
_VC = 8192


def _permute(v):
    return (
        ((v >> 13) << 13)
        | ((v & 2047) << 2)
        | ((v & 8191) >> 11)
    )


def _tc_index_prep(x, B, L, LA, LB):
    BM = 512

    def body(x_ref, a_ref, b_ref):
        xv = x_ref[...]
        a_ref[...] = _permute(xv[:, :LA])
        tail = _permute(xv[:, LA:L])
        b_ref[...] = jnp.concatenate(
            [tail, jnp.zeros((BM, LA + LB - L), jnp.int32)], axis=1)

    return pl.pallas_call(
        body,
        grid=(B // BM,),
        in_specs=[pl.BlockSpec((BM, L), lambda i: (i, 0))],
        out_specs=[
            pl.BlockSpec((BM, LA), lambda i: (i, 0)),
            pl.BlockSpec((BM, LB), lambda i: (i, 0)),
        ],
        out_shape=[
            jax.ShapeDtypeStruct((B, LA), jnp.int32),
            jax.ShapeDtypeStruct((B, LB), jnp.int32),
        ],
    )(x)


def _tc_scale_transpose(tt, w, V, D):
    VC = _VC
    NB = pl.cdiv(V, VC)
    CH = VC // 4

    def body(t_ref, w_ref, eye_ref, o_ref):
        scaled = t_ref[...] * w_ref[...][None, :]
        s_t = lax.dot_general(scaled, eye_ref[...], (((0,), (0,)), ((), ())),
                              preferred_element_type=jnp.float32)
        o_ref[...] = jnp.concatenate(
            [s_t[j * CH:(j + 1) * CH] for j in range(4)], axis=1)

    return pl.pallas_call(
        body,
        grid=(NB,),
        in_specs=[
            pl.BlockSpec((D, VC), lambda i: (0, i)),
            pl.BlockSpec((VC,), lambda i: (i,)),
            pl.BlockSpec((D, D), lambda i: (0, 0)),
        ],
        out_specs=pl.BlockSpec((VC * D // 128, 128), lambda i: (i, 0)),
        out_shape=jax.ShapeDtypeStruct((NB * VC * D // 128, 128),
                                       jnp.float32),
    )(tt, w, jnp.eye(D, dtype=jnp.float32))


def _sc_pooled_embedding(xa, xb, tvd, B, D, LA, LB):
    RT = _ROWS_PER_TILE
    LP = LA + LB
    TOK = RT * LP
    b_per_w = B // _N_WORKERS
    tiles_per_w = b_per_w // RT
    mesh = plsc.VectorSubcoreMesh(core_axis_name="c", subcore_axis_name="s")

    @functools.partial(
        pl.kernel,
        out_type=jax.ShapeDtypeStruct((B, D), jnp.float32),
        mesh=mesh,
        compiler_params=pltpu.CompilerParams(use_tc_tiling_on_sc=False),
        scratch_types=[
            pltpu.VMEM((2, RT, LA), jnp.int32),
            pltpu.VMEM((2, RT, LB), jnp.int32),
            pltpu.VMEM((2, TOK, D), jnp.float32),
            pltpu.VMEM((b_per_w, D), jnp.float32),
            pltpu.SemaphoreType.DMA,
            pltpu.SemaphoreType.DMA,
        ],
    )
    def body(xa_hbm, xb_hbm, table_hbm, out_hbm,
             xa_v, xb_v, rows_v, out_v, gsem0, gsem1):
        sid = lax.axis_index("s")
        wid = lax.axis_index("c") * (_N_WORKERS // 2) + sid
        tbase = wid * tiles_per_w

        def descs(par, sem):
            cps = []
            for r in range(RT):
                cps.append(pltpu.make_async_copy(
                    table_hbm.at[xa_v.at[par, r]],
                    rows_v.at[par, pl.ds(r * LP, LA)], sem))
                cps.append(pltpu.make_async_copy(
                    table_hbm.at[xb_v.at[par, r]],
                    rows_v.at[par, pl.ds(r * LP + LA, LB)], sem))
            return cps

        def fire(t, par, sem):
            r0 = (tbase + t) * RT
            pltpu.sync_copy(xa_hbm.at[pl.ds(r0, RT)], xa_v.at[par])
            pltpu.sync_copy(xb_hbm.at[pl.ds(r0, RT)], xb_v.at[par])
            for c in descs(par, sem):
                c.start()

        def drain(par, sem):
            for c in descs(par, sem):
                c.wait()

        fire(0, 0, gsem0)

        def do_tile(t, carry):
            par = lax.rem(t, 2)

            @pl.when(t + 1 < tiles_per_w)
            def _():
                lax.cond(par == 0,
                         lambda: fire(t + 1, 1, gsem1),
                         lambda: fire(t + 1, 0, gsem0))

            lax.cond(par == 0,
                     lambda: drain(0, gsem0),
                     lambda: drain(1, gsem1))

            ii = lax.iota(jnp.int32, 16)
            for r in range(RT):
                base = r * LP
                cnt = jnp.zeros((16,), jnp.float32)
                for i in range(LP // 16):
                    off = i * 16
                    if off < LA:
                        xv = xa_v[par, r, pl.ds(off, 16)]
                    else:
                        xv = xb_v[par, r, pl.ds(off - LA, 16)]
                    cnt = cnt + jnp.where(xv != 0, 1.0, 0.0)
                for sh in (8, 4, 2, 1):
                    cnt = cnt + cnt.at[ii ^ sh].get(mode="promise_in_bounds")

                accs = [jnp.zeros((16,), jnp.float32) for _ in range(D // 16)]
                for l in range(LP):
                    for j in range(D // 16):
                        accs[j] = accs[j] + rows_v[par, base + l,
                                                   pl.ds(j * 16, 16)]
                inv = 1.0 / (cnt + 1e-16)
                for j in range(D // 16):
                    out_v[t * RT + r, pl.ds(j * 16, 16)] = accs[j] * inv
            return carry

        lax.fori_loop(0, tiles_per_w, do_tile, 0)
        pltpu.sync_copy(out_v, out_hbm.at[pl.ds(wid * b_per_w, b_per_w)])

    return body(xa, xb, tvd)


def _tc_linear(embedded, wt, b2, B, D, OUT):
    BM = 512

    def mm_body(e_ref, w_ref, b_ref, o_ref):
        o_ref[...] = (
            jnp.dot(e_ref[...], w_ref[...], preferred_element_type=jnp.float32)
            + b_ref[...]
        )

    return pl.pallas_call(
        mm_body,
        grid=(B // BM,),
        in_specs=[
            pl.BlockSpec((BM, D), lambda i: (i, 0)),
            pl.BlockSpec((D, OUT), lambda i: (0, 0)),
            pl.BlockSpec((1, OUT), lambda i: (0, 0)),
        ],
        out_specs=pl.BlockSpec((BM, OUT), lambda i: (i, 0)),
        out_shape=jax.ShapeDtypeStruct((B, OUT), jnp.float32),
    )(embedded, wt, b2)


def kernel(x, table, w, W_out, b_out):
    B, L = x.shape
    V, D = table.shape
    OUT = W_out.shape[0]
    LA = _LA
    LB = ((L - LA + 15) // 16) * 16

    x = x.astype(jnp.int32)
    xa, xb = _tc_index_prep(x, B, L, LA, LB)

    tf128 = _tc_scale_transpose(table.T, w, V, D)
    vp = tf128.shape[0] * 128 // D
    tvd = tf128.reshape(vp, D)

    embedded = _sc_pooled_embedding(xa, xb, tvd, B, D, LA, LB)
    out = _tc_linear(embedded, W_out.T, b_out[None, :], B, D, OUT)
    return (out, embedded)


# speedup vs baseline: 1.2936x; 1.2936x over previous
import functools

import jax
import jax.numpy as jnp
from jax import lax
from jax.experimental import pallas as pl
from jax.experimental.pallas import tpu as pltpu
from jax.experimental.pallas import tpu_sc as plsc

_N_WORKERS = 32
_ROWS_PER_TILE = 8
_LA = 128
_VC = 8192


def _permute(v):
    return (
        ((v >> 13) << 13)
        | ((v & 1023) << 3)
        | ((v & 8191) >> 10)
    )


def _tc_index_prep(x, B, L, LA, LB):
    BM = 512

    def body(x_ref, a_ref, b_ref):
        xv = x_ref[...]
        a_ref[...] = _permute(xv[:, :LA])
        tail = _permute(xv[:, LA:L])
        b_ref[...] = jnp.concatenate(
            [tail, jnp.zeros((BM, LA + LB - L), jnp.int32)], axis=1)

    return pl.pallas_call(
        body,
        grid=(B // BM,),
        in_specs=[pl.BlockSpec((BM, L), lambda i: (i, 0))],
        out_specs=[
            pl.BlockSpec((BM, LA), lambda i: (i, 0)),
            pl.BlockSpec((BM, LB), lambda i: (i, 0)),
        ],
        out_shape=[
            jax.ShapeDtypeStruct((B, LA), jnp.int32),
            jax.ShapeDtypeStruct((B, LB), jnp.int32),
        ],
    )(x)


def _tc_scale_transpose(tt, w, V, D):
    VC = _VC
    NB = pl.cdiv(V, VC)
    CH = VC // 8

    def body(t_ref, w_ref, se_ref, so_ref, o_ref):
        scaled = t_ref[...] * w_ref[...][None, :]
        dn = (((0,), (0,)), ((), ()))
        s_e = lax.dot_general(scaled, se_ref[...], dn,
                              preferred_element_type=jnp.float32)
        s_o = lax.dot_general(scaled, so_ref[...], dn,
                              preferred_element_type=jnp.float32)
        be = lax.bitcast_convert_type(
            s_e.astype(jnp.bfloat16).astype(jnp.float32), jnp.int32)
        bo = lax.bitcast_convert_type(
            s_o.astype(jnp.bfloat16).astype(jnp.float32), jnp.int32)
        u = lax.shift_right_logical(be, 16) | (bo & jnp.int32(-65536))
        o_ref[...] = jnp.concatenate(
            [u[j * CH:(j + 1) * CH] for j in range(8)], axis=1)

    sel_e = jnp.zeros((D, D // 2), jnp.float32)
    sel_e = sel_e.at[2 * jnp.arange(D // 2), jnp.arange(D // 2)].set(1.0)
    sel_o = jnp.zeros((D, D // 2), jnp.float32)
    sel_o = sel_o.at[2 * jnp.arange(D // 2) + 1, jnp.arange(D // 2)].set(1.0)

    return pl.pallas_call(
        body,
        grid=(NB,),
        in_specs=[
            pl.BlockSpec((D, VC), lambda i: (0, i)),
            pl.BlockSpec((VC,), lambda i: (i,)),
            pl.BlockSpec((D, D // 2), lambda i: (0, 0)),
            pl.BlockSpec((D, D // 2), lambda i: (0, 0)),
        ],
        out_specs=pl.BlockSpec((VC * D // 256, 128), lambda i: (i, 0)),
        out_shape=jax.ShapeDtypeStruct((NB * VC * D // 256, 128),
                                       jnp.int32),
    )(tt, w, sel_e, sel_o)


def _sc_pooled_embedding(xa, xb, tvd, B, D, LA, LB):
    RT = _ROWS_PER_TILE
    LP = LA + LB
    TOK = RT * LP
    b_per_w = B // _N_WORKERS
    tiles_per_w = b_per_w // RT
    mesh = plsc.VectorSubcoreMesh(core_axis_name="c", subcore_axis_name="s")

    @functools.partial(
        pl.kernel,
        out_type=jax.ShapeDtypeStruct((B, D), jnp.float32),
        mesh=mesh,
        compiler_params=pltpu.CompilerParams(use_tc_tiling_on_sc=False,
                                             needs_layout_passes=False),
        scratch_types=[
            pltpu.VMEM((2, RT, LA), jnp.int32),
            pltpu.VMEM((2, RT, LB), jnp.int32),
            pltpu.VMEM((2, TOK, D // 2), jnp.int32),
            pltpu.VMEM((b_per_w, D), jnp.float32),
            pltpu.SemaphoreType.DMA,
            pltpu.SemaphoreType.DMA,
        ],
    )
    def body(xa_hbm, xb_hbm, table_hbm, out_hbm,
             xa_v, xb_v, rows_v, out_v, gsem0, gsem1):
        sid = lax.axis_index("s")
        wid = lax.axis_index("c") * (_N_WORKERS // 2) + sid
        tbase = wid * tiles_per_w

        def descs(par, sem):
            cps = []
            for r in range(RT):
                cps.append(pltpu.make_async_copy(
                    table_hbm.at[xa_v.at[par, r]],
                    rows_v.at[par, pl.ds(r * LP, LA)], sem))
                cps.append(pltpu.make_async_copy(
                    table_hbm.at[xb_v.at[par, r]],
                    rows_v.at[par, pl.ds(r * LP + LA, LB)], sem))
            return cps

        def fire(t, par, sem):
            r0 = (tbase + t) * RT
            pltpu.sync_copy(xa_hbm.at[pl.ds(r0, RT)], xa_v.at[par])
            pltpu.sync_copy(xb_hbm.at[pl.ds(r0, RT)], xb_v.at[par])
            for c in descs(par, sem):
                c.start()

        def drain(par, sem):
            for c in descs(par, sem):
                c.wait()

        fire(0, 0, gsem0)

        def do_tile(t, carry):
            par = lax.rem(t, 2)

            @pl.when(t + 1 < tiles_per_w)
            def _():
                lax.cond(par == 0,
                         lambda: fire(t + 1, 1, gsem1),
                         lambda: fire(t + 1, 0, gsem0))

            lax.cond(par == 0,
                     lambda: drain(0, gsem0),
                     lambda: drain(1, gsem1))

            ii = lax.iota(jnp.int32, 16)
            for r in range(RT):
                base = r * LP
                cnt = jnp.zeros((16,), jnp.float32)
                for i in range(LP // 16):
                    off = i * 16
                    if off < LA:
                        xv = xa_v[par, r, pl.ds(off, 16)]
                    else:
                        xv = xb_v[par, r, pl.ds(off - LA, 16)]
                    cnt = cnt + jnp.where(xv != 0, 1.0, 0.0)
                for sh in (8, 4, 2, 1):
                    cnt = cnt + cnt.at[ii ^ sh].get(mode="promise_in_bounds")

                acc_e = jnp.zeros((16,), jnp.float32)
                acc_o = jnp.zeros((16,), jnp.float32)
                for l in range(LP):
                    u = rows_v[par, base + l, :]
                    acc_e = acc_e + plsc.bitcast(u << 16, jnp.float32)
                    acc_o = acc_o + plsc.bitcast(
                        u & jnp.int32(-65536), jnp.float32)
                inv = 1.0 / (cnt + 1e-16)
                acc_e = acc_e * inv
                acc_o = acc_o * inv
                ev = (ii & 1) == 0
                for j in range(D // 16):
                    k = (ii >> 1) + j * 8
                    ee = acc_e.at[k].get(mode="promise_in_bounds")
                    oo = acc_o.at[k].get(mode="promise_in_bounds")
                    out_v[t * RT + r, pl.ds(j * 16, 16)] = jnp.where(
                        ev, ee, oo)
            return carry

        lax.fori_loop(0, tiles_per_w, do_tile, 0)
        pltpu.sync_copy(out_v, out_hbm.at[pl.ds(wid * b_per_w, b_per_w)])

    return body(xa, xb, tvd)


def _tc_linear(embedded, wt, b2, B, D, OUT):
    BM = 512

    def mm_body(e_ref, w_ref, b_ref, o_ref):
        o_ref[...] = (
            jnp.dot(e_ref[...], w_ref[...], preferred_element_type=jnp.float32)
            + b_ref[...]
        )

    return pl.pallas_call(
        mm_body,
        grid=(B // BM,),
        in_specs=[
            pl.BlockSpec((BM, D), lambda i: (i, 0)),
            pl.BlockSpec((D, OUT), lambda i: (0, 0)),
            pl.BlockSpec((1, OUT), lambda i: (0, 0)),
        ],
        out_specs=pl.BlockSpec((BM, OUT), lambda i: (i, 0)),
        out_shape=jax.ShapeDtypeStruct((B, OUT), jnp.float32),
    )(embedded, wt, b2)


def kernel(x, table, w, W_out, b_out):
    B, L = x.shape
    V, D = table.shape
    OUT = W_out.shape[0]
    LA = _LA
    LB = ((L - LA + 15) // 16) * 16

    x = x.astype(jnp.int32)
    xa, xb = _tc_index_prep(x, B, L, LA, LB)

    tf128 = _tc_scale_transpose(table.T, w, V, D)
    vp = tf128.shape[0] * 256 // D
    tvd = tf128.reshape(vp, D // 2)

    embedded = _sc_pooled_embedding(xa, xb, tvd, B, D, LA, LB)
    out = _tc_linear(embedded, W_out.T, b_out[None, :], B, D, OUT)
    return (out, embedded)
